# baseline (speedup 1.0000x reference)
import jax
import jax.numpy as jnp
from jax import lax
from jax.experimental import pallas as pl
from jax.experimental.pallas import tpu as pltpu
from jax.experimental.pallas import tpu_sc as plsc

_N_NODES = 10000
_N_EDGES = 320000
_D = 128
_L = 16
_NC = 2
_NS = 16
_NW = _NC * _NS
_K = 64
_N_CHUNKS = _N_EDGES // _K
_BASE_CHUNKS = _N_CHUNKS // _NW
_EXTRA_W = _N_CHUNKS - _BASE_CHUNKS * _NW
_MAXC = _BASE_CHUNKS + 1
_UNROLL = 8


def _body(ah_hbm, bh_hbm, us_hbm, vs_hbm, ws_hbm, out_hbm,
          idx_u, idx_v, ws_v,
          rows_u0, rows_u1, rows_u2, rows_u3, rows_u4, rows_u5,
          rows_v0, rows_v1, rows_v2, rows_v3, rows_v4, rows_v5,
          out0, out1, out2, out3, out4, out5,
          sem_idx, sem_rows0, sem_rows1, sem_rows2, sem_rows3, sem_rows4,
          sem_rows5, sem_out):
    wid = lax.axis_index("s") * _NC + lax.axis_index("c")
    lane = lax.iota(jnp.int32, _L)
    rows_u = (rows_u0, rows_u1, rows_u2, rows_u3, rows_u4, rows_u5)
    rows_v = (rows_v0, rows_v1, rows_v2, rows_v3, rows_v4, rows_v5)
    out_b = (out0, out1, out2, out3, out4, out5)
    sem_rows = (sem_rows0, sem_rows1, sem_rows2, sem_rows3, sem_rows4,
                sem_rows5)

    has_extra = wid < _EXTRA_W
    n_chunks = jnp.where(has_extra, _MAXC, _BASE_CHUNKS)
    start = _BASE_CHUNKS * wid + jnp.minimum(wid, _EXTRA_W)
    ebase = start * _K

    nmain = _BASE_CHUNKS * _K
    pltpu.make_async_copy(
        us_hbm.at[pl.ds(ebase, nmain)], idx_u.at[pl.ds(0, nmain)], sem_idx
    ).start()
    pltpu.make_async_copy(
        vs_hbm.at[pl.ds(ebase, nmain)], idx_v.at[pl.ds(0, nmain)], sem_idx
    ).start()
    pltpu.make_async_copy(
        ws_hbm.at[pl.ds(ebase, nmain)], ws_v.at[pl.ds(0, nmain)], sem_idx
    ).start()

    @pl.when(has_extra)
    def _():
        for hbm, vmem in ((us_hbm, idx_u), (vs_hbm, idx_v), (ws_hbm, ws_v)):
            pltpu.make_async_copy(
                hbm.at[pl.ds(ebase + nmain, _K)],
                vmem.at[pl.ds(nmain, _K)], sem_idx,
            ).start()

    def row_copies(m, p):
        off = m * _K
        return (
            pltpu.make_async_copy(
                ah_hbm.at[idx_u.at[pl.ds(off, _K)]], rows_u[p], sem_rows[p]),
            pltpu.make_async_copy(
                bh_hbm.at[idx_v.at[pl.ds(off, _K)]], rows_v[p], sem_rows[p]),
        )

    def compute(m, p):
        off = m * _K
        for g in range(_K // _L):
            row = lane + (g * _L)

            def dim_body(d0, carry2):
                acc0, acc1 = carry2
                for k in range(_UNROLL):
                    col = (lane + (d0 * _UNROLL + k)) & (_D - 1)
                    gu = plsc.load_gather(rows_u[p], [row, col])
                    gv = plsc.load_gather(rows_v[p], [row, col])
                    cmb = gu + gv
                    if k % 2 == 0:
                        acc0 = acc0 + cmb * cmb
                    else:
                        acc1 = acc1 + cmb * cmb
                return acc0, acc1

            zero = jnp.zeros((_L,), jnp.float32)
            acc0, acc1 = lax.fori_loop(0, _D // _UNROLL, dim_body,
                                       (zero, zero))
            acc = acc0 + acc1
            w = ws_v[pl.ds(off + g * _L, _L)]
            out_b[p][pl.ds(g * _L, _L)] = w / (1.0 + jnp.exp(10.0 - acc))

    drain = pltpu.make_async_copy(
        us_hbm.at[pl.ds(ebase, nmain)], idx_u.at[pl.ds(0, nmain)], sem_idx)
    for _i in range(3):
        drain.wait()

    @pl.when(has_extra)
    def _():
        d = pltpu.make_async_copy(
            us_hbm.at[pl.ds(ebase + nmain, _K)],
            idx_u.at[pl.ds(nmain, _K)], sem_idx)
        for _i in range(3):
            d.wait()

    def out_copy(m, p):
        return pltpu.make_async_copy(
            out_b[p], out_hbm.at[pl.ds(ebase + m * _K, _K)], sem_out)

    _DEPTH = 6
    for q in range(_DEPTH - 1):
        for cp in row_copies(q, q):
            cp.start()

    def process(m, p):
        @pl.when(m + _DEPTH - 1 < n_chunks)
        def _():
            for cp in row_copies(m + _DEPTH - 1, (p + _DEPTH - 1) % _DEPTH):
                cp.start()

        live = m < n_chunks

        @pl.when(jnp.logical_and(live, m >= _DEPTH))
        def _():
            out_copy(m - _DEPTH, p).wait()

        @pl.when(live)
        def _():
            for cp in row_copies(m, p):
                cp.wait()
            compute(m, p)
            out_copy(m, p).start()

    def ring_body(i6, carry):
        for q in range(_DEPTH):
            process(i6 * _DEPTH + q, q)
        return carry

    lax.fori_loop(0, (_MAXC + _DEPTH - 1) // _DEPTH, ring_body, 0)

    for p in range(_DEPTH):
        out_copy(0, p).wait()


@jax.jit
def _scorer(ah, bh, us, vs, ws):
    mesh = plsc.VectorSubcoreMesh(
        core_axis_name="c", subcore_axis_name="s",
        num_cores=_NC, num_subcores=_NS)
    return pl.kernel(
        _body,
        out_type=jax.ShapeDtypeStruct((_N_EDGES,), jnp.float32),
        mesh=mesh,
        compiler_params=pltpu.CompilerParams(needs_layout_passes=False),
        scratch_types=[
            pltpu.VMEM((_MAXC * _K,), jnp.int32),
            pltpu.VMEM((_MAXC * _K,), jnp.int32),
            pltpu.VMEM((_MAXC * _K,), jnp.float32),
            *[pltpu.VMEM((_K, _D), jnp.float32) for _ in range(6)],
            *[pltpu.VMEM((_K, _D), jnp.float32) for _ in range(6)],
            *[pltpu.VMEM((_K,), jnp.float32) for _ in range(6)],
            pltpu.SemaphoreType.DMA,
            *[pltpu.SemaphoreType.DMA for _ in range(6)],
            pltpu.SemaphoreType.DMA,
        ],
    )(ah, bh, us, vs, ws)


def kernel(h, us, vs, ws, a, b):
    return _scorer(h * a, h * b, us, vs, ws)


# speedup vs baseline: 1.0009x; 1.0009x over previous
import jax
import jax.numpy as jnp
from jax import lax
from jax.experimental import pallas as pl
from jax.experimental.pallas import tpu as pltpu
from jax.experimental.pallas import tpu_sc as plsc

_N_NODES = 10000
_N_EDGES = 320000
_D = 128
_L = 16
_NC = 2
_NS = 16
_NW = _NC * _NS
_K = 128
_N_CHUNKS = _N_EDGES // _K
_BASE_CHUNKS = _N_CHUNKS // _NW
_EXTRA_W = _N_CHUNKS - _BASE_CHUNKS * _NW
_MAXC = _BASE_CHUNKS + 1
_UNROLL = 8


def _body(ah_hbm, bh_hbm, us_hbm, vs_hbm, ws_hbm, out_hbm,
          idx_u, idx_v, ws_v,
          rows_u0, rows_u1, rows_u2, rows_v0, rows_v1, rows_v2,
          out0, out1, out2,
          sem_idx, sem_rows0, sem_rows1, sem_rows2, sem_out):
    wid = lax.axis_index("s") * _NC + lax.axis_index("c")
    lane = lax.iota(jnp.int32, _L)
    rows_u = (rows_u0, rows_u1, rows_u2)
    rows_v = (rows_v0, rows_v1, rows_v2)
    out_b = (out0, out1, out2)
    sem_rows = (sem_rows0, sem_rows1, sem_rows2)

    has_extra = wid < _EXTRA_W
    n_chunks = jnp.where(has_extra, _MAXC, _BASE_CHUNKS)
    start = _BASE_CHUNKS * wid + jnp.minimum(wid, _EXTRA_W)
    ebase = start * _K

    nmain = _BASE_CHUNKS * _K
    pltpu.make_async_copy(
        us_hbm.at[pl.ds(ebase, nmain)], idx_u.at[pl.ds(0, nmain)], sem_idx
    ).start()
    pltpu.make_async_copy(
        vs_hbm.at[pl.ds(ebase, nmain)], idx_v.at[pl.ds(0, nmain)], sem_idx
    ).start()
    pltpu.make_async_copy(
        ws_hbm.at[pl.ds(ebase, nmain)], ws_v.at[pl.ds(0, nmain)], sem_idx
    ).start()

    @pl.when(has_extra)
    def _():
        for hbm, vmem in ((us_hbm, idx_u), (vs_hbm, idx_v), (ws_hbm, ws_v)):
            pltpu.make_async_copy(
                hbm.at[pl.ds(ebase + nmain, _K)],
                vmem.at[pl.ds(nmain, _K)], sem_idx,
            ).start()

    def row_copies(m, p):
        off = m * _K
        return (
            pltpu.make_async_copy(
                ah_hbm.at[idx_u.at[pl.ds(off, _K)]], rows_u[p], sem_rows[p]),
            pltpu.make_async_copy(
                bh_hbm.at[idx_v.at[pl.ds(off, _K)]], rows_v[p], sem_rows[p]),
        )

    def compute(m, p):
        off = m * _K
        for g in range(_K // _L):
            row = lane + (g * _L)

            def dim_body(d0, carry2):
                acc0, acc1 = carry2
                for k in range(_UNROLL):
                    col = (lane + (d0 * _UNROLL + k)) & (_D - 1)
                    gu = plsc.load_gather(rows_u[p], [row, col])
                    gv = plsc.load_gather(rows_v[p], [row, col])
                    cmb = gu + gv
                    if k % 2 == 0:
                        acc0 = acc0 + cmb * cmb
                    else:
                        acc1 = acc1 + cmb * cmb
                return acc0, acc1

            zero = jnp.zeros((_L,), jnp.float32)
            acc0, acc1 = lax.fori_loop(0, _D // _UNROLL, dim_body,
                                       (zero, zero))
            acc = acc0 + acc1
            w = ws_v[pl.ds(off + g * _L, _L)]
            out_b[p][pl.ds(g * _L, _L)] = w / (1.0 + jnp.exp(10.0 - acc))

    drain = pltpu.make_async_copy(
        us_hbm.at[pl.ds(ebase, nmain)], idx_u.at[pl.ds(0, nmain)], sem_idx)
    for _i in range(3):
        drain.wait()

    @pl.when(has_extra)
    def _():
        d = pltpu.make_async_copy(
            us_hbm.at[pl.ds(ebase + nmain, _K)],
            idx_u.at[pl.ds(nmain, _K)], sem_idx)
        for _i in range(3):
            d.wait()

    def out_copy(m, p):
        return pltpu.make_async_copy(
            out_b[p], out_hbm.at[pl.ds(ebase + m * _K, _K)], sem_out)

    for cp in row_copies(0, 0):
        cp.start()
    for cp in row_copies(1, 1):
        cp.start()

    def process(m, p):
        @pl.when(m + 2 < n_chunks)
        def _():
            for cp in row_copies(m + 2, (p + 2) % 3):
                cp.start()

        live = m < n_chunks

        @pl.when(jnp.logical_and(live, m >= 3))
        def _():
            out_copy(m - 3, p).wait()

        @pl.when(live)
        def _():
            for cp in row_copies(m, p):
                cp.wait()
            compute(m, p)
            out_copy(m, p).start()

    def triple_body(i3, carry):
        process(i3 * 3, 0)
        process(i3 * 3 + 1, 1)
        process(i3 * 3 + 2, 2)
        return carry

    lax.fori_loop(0, (_MAXC + 2) // 3, triple_body, 0)

    for p in range(3):
        out_copy(0, p).wait()


@jax.jit
def _scorer(ah, bh, us, vs, ws):
    mesh = plsc.VectorSubcoreMesh(
        core_axis_name="c", subcore_axis_name="s",
        num_cores=_NC, num_subcores=_NS)
    return pl.kernel(
        _body,
        out_type=jax.ShapeDtypeStruct((_N_EDGES,), jnp.float32),
        mesh=mesh,
        compiler_params=pltpu.CompilerParams(needs_layout_passes=False),
        scratch_types=[
            pltpu.VMEM((_MAXC * _K,), jnp.int32),
            pltpu.VMEM((_MAXC * _K,), jnp.int32),
            pltpu.VMEM((_MAXC * _K,), jnp.float32),
            pltpu.VMEM((_K, _D), jnp.float32),
            pltpu.VMEM((_K, _D), jnp.float32),
            pltpu.VMEM((_K, _D), jnp.float32),
            pltpu.VMEM((_K, _D), jnp.float32),
            pltpu.VMEM((_K, _D), jnp.float32),
            pltpu.VMEM((_K, _D), jnp.float32),
            pltpu.VMEM((_K,), jnp.float32),
            pltpu.VMEM((_K,), jnp.float32),
            pltpu.VMEM((_K,), jnp.float32),
            pltpu.SemaphoreType.DMA,
            pltpu.SemaphoreType.DMA,
            pltpu.SemaphoreType.DMA,
            pltpu.SemaphoreType.DMA,
            pltpu.SemaphoreType.DMA,
        ],
    )(ah, bh, us, vs, ws)


def kernel(h, us, vs, ws, a, b):
    return _scorer(h * a, h * b, us, vs, ws)
